# baseline (speedup 1.0000x reference)
import functools
import math

import jax
import jax.numpy as jnp
import numpy as np
from jax import lax
from jax.experimental import pallas as pl
from jax.experimental.pallas import tpu as pltpu
from jax.experimental.pallas import tpu_sc as plsc

_N_NODES = 10000
_N_EDGES = 320000
_D = 128

_BE = 12288
_GRID = -(-_N_EDGES // _BE)

_NSUB = 16
_CH = _N_EDGES // _NSUB
_NCHUNK = _CH // 16
_NPAD = 10240
_CPT = _NPAD // _NSUB


_F32 = jnp.float32
_DN_K1 = (((0,), (0,)), ((), ()))
_DN_RT = (((1,), (1,)), ((), ()))


def _att_body(z_ref, wq_ref, wk_ref, o0_ref, o1_ref, mx_ref):
    i = pl.program_id(0)
    z = z_ref[...]
    scale = 1.0 / math.sqrt(_D)
    a0w = lax.dot_general(wq_ref[0], wk_ref[0], _DN_K1,
                          preferred_element_type=_F32) * scale
    a1w = lax.dot_general(wq_ref[1], wk_ref[1], _DN_K1,
                          preferred_element_type=_F32) * scale
    p = jnp.concatenate([a0w, a1w], axis=1)
    t = jnp.dot(z, p, preferred_element_type=_F32)
    zz = jnp.concatenate([z, z], axis=1)
    hsel = lax.broadcasted_iota(jnp.int32, (2 * _D, 2), 0) // _D
    csel = lax.broadcasted_iota(jnp.int32, (2 * _D, 2), 1)
    sel = jnp.where(hsel == csel, 1.0, 0.0).astype(_F32)
    a01 = jnp.dot(t * zz, sel, preferred_element_type=_F32)
    a_t = a01.T
    o0_ref[...] = a_t[0]
    o1_ref[...] = a_t[1]
    rows = lax.broadcasted_iota(jnp.int32, (_BE, 2), 0)
    valid = _N_EDGES - i * _BE
    a01m = jnp.where(rows < valid, a01, -3.0e38)
    mfull = jnp.full((1, 1, _D), jnp.max(a01m), jnp.float32)

    @pl.when(i == 0)
    def _():
        mx_ref[...] = mfull

    @pl.when(i > 0)
    def _():
        mx_ref[...] = jnp.maximum(mx_ref[...], mfull)


def _out_body(z_ref, w0_ref, w1_ref, wv_ref, w1w_ref, b1_ref, w2w_ref,
              b2_ref, o_ref):
    _BF = jnp.bfloat16
    z = z_ref[...].astype(_BF)
    wv = wv_ref[...].astype(_BF)
    v0 = lax.dot_general(z, wv[0], _DN_RT, preferred_element_type=_F32)
    v1 = lax.dot_general(z, wv[1], _DN_RT, preferred_element_type=_F32)
    w_t = jnp.stack([w0_ref[...], w1_ref[...]], axis=0).T
    zu = (w_t[:, 0:1] * v0 + w_t[:, 1:2] * v1).astype(_BF)
    h = lax.dot_general(zu, w1w_ref[...].astype(_BF), _DN_RT,
                        preferred_element_type=_F32)
    h = h + b1_ref[...][None, :]
    h = h * jax.nn.sigmoid(h)
    o = lax.dot_general(h.astype(_BF), w2w_ref[...].astype(_BF), _DN_RT,
                        preferred_element_type=_F32)
    o_ref[...] = o + b2_ref[...][None, :]


def _sc_softmax_body(att0_hbm, att1_hbm, row_hbm, gmax_hbm,
                     w0_hbm, w1_hbm,
                     att_v, idx_v, denom_v, red_v, tot_v, gmax_v,
                     partials_sh, total_sh):
    c = lax.axis_index("c")
    s = lax.axis_index("s")
    base = s * _CH

    pltpu.sync_copy(gmax_hbm.at[0, 0, pl.ds(0, 16)], gmax_v)

    @pl.when(c == 0)
    def _():
        pltpu.sync_copy(att0_hbm.at[pl.ds(base, _CH)], att_v)

    @pl.when(c == 1)
    def _():
        pltpu.sync_copy(att1_hbm.at[pl.ds(base, _CH)], att_v)

    pltpu.sync_copy(row_hbm.at[pl.ds(base, _CH)], idx_v)

    @plsc.parallel_loop(0, _NPAD, step=16)
    def _(i):
        denom_v[pl.ds(pl.multiple_of(i, 16), 16)] = jnp.zeros(
            (16,), jnp.float32)

    gm = gmax_v[...]

    @plsc.parallel_loop(0, _CH, step=16)
    def _(i):
        off = pl.multiple_of(i, 16)
        idxv = idx_v[pl.ds(off, 16)]
        ev = jnp.exp(att_v[pl.ds(off, 16)] - gm)
        att_v[pl.ds(off, 16)] = ev
        plsc.addupdate_scatter(denom_v, [idxv], ev)

    pltpu.sync_copy(denom_v, partials_sh.at[s])
    plsc.subcore_barrier()
    colbase = s * _CPT
    pltpu.sync_copy(partials_sh.at[:, pl.ds(colbase, _CPT)], red_v)

    @plsc.parallel_loop(0, _CPT, step=16)
    def _(j):
        off = pl.multiple_of(j, 16)
        acc = red_v[0, pl.ds(off, 16)]
        for r in range(1, _NSUB):
            acc = acc + red_v[r, pl.ds(off, 16)]
        tot_v[pl.ds(off, 16)] = acc
    pltpu.sync_copy(tot_v, total_sh.at[pl.ds(colbase, _CPT)])
    plsc.subcore_barrier()
    pltpu.sync_copy(total_sh, denom_v)

    @plsc.parallel_loop(0, _CH, step=16)
    def _(i):
        off = pl.multiple_of(i, 16)
        idxv = idx_v[pl.ds(off, 16)]
        ev = att_v[pl.ds(off, 16)]
        dv = plsc.load_gather(denom_v, [idxv])
        att_v[pl.ds(off, 16)] = ev / dv

    @pl.when(c == 0)
    def _():
        pltpu.sync_copy(att_v, w0_hbm.at[pl.ds(base, _CH)])

    @pl.when(c == 1)
    def _():
        pltpu.sync_copy(att_v, w1_hbm.at[pl.ds(base, _CH)])


@functools.cache
def _sc_softmax():
  return functools.partial(
    pl.kernel,
    out_type=[jax.ShapeDtypeStruct((_N_EDGES,), jnp.float32),
              jax.ShapeDtypeStruct((_N_EDGES,), jnp.float32)],
    mesh=plsc.VectorSubcoreMesh(core_axis_name="c", subcore_axis_name="s",
                                num_cores=2, num_subcores=_NSUB),
    compiler_params=pltpu.CompilerParams(needs_layout_passes=False),
    scratch_types=[
        pltpu.VMEM((_CH,), jnp.float32),
        pltpu.VMEM((_CH,), jnp.int32),
        pltpu.VMEM((_NPAD,), jnp.float32),
        pltpu.VMEM((_NSUB, _CPT), jnp.float32),
        pltpu.VMEM((_CPT,), jnp.float32),
        pltpu.VMEM((16,), jnp.float32),
        pltpu.VMEM_SHARED((_NSUB, _NPAD), jnp.float32),
        pltpu.VMEM_SHARED((_NPAD,), jnp.float32),
    ],
  )(_sc_softmax_body)


def kernel(Z, edges, Wq, Wk, Wv, W1, b1, W2, b2):
    att0, att1, mx = pl.pallas_call(
        _att_body,
        grid=(_GRID,),
        in_specs=[
            pl.BlockSpec((_BE, _D), lambda i: (i, 0)),
            pl.BlockSpec((2, _D, _D), lambda i: (0, 0, 0)),
            pl.BlockSpec((2, _D, _D), lambda i: (0, 0, 0)),
        ],
        out_specs=[
            pl.BlockSpec((_BE,), lambda i: (i,)),
            pl.BlockSpec((_BE,), lambda i: (i,)),
            pl.BlockSpec((1, 1, _D), lambda i: (0, 0, 0)),
        ],
        out_shape=[
            jax.ShapeDtypeStruct((_N_EDGES,), jnp.float32),
            jax.ShapeDtypeStruct((_N_EDGES,), jnp.float32),
            jax.ShapeDtypeStruct((1, 1, _D), jnp.float32),
        ],
    )(Z, Wq, Wk)

    row2e = edges.astype(jnp.int32).reshape(2 * _N_EDGES)
    w0, w1 = _sc_softmax()(att0, att1, row2e, mx)

    out = pl.pallas_call(
        _out_body,
        grid=(_GRID,),
        in_specs=[
            pl.BlockSpec((_BE, _D), lambda i: (i, 0)),
            pl.BlockSpec((_BE,), lambda i: (i,)),
            pl.BlockSpec((_BE,), lambda i: (i,)),
            pl.BlockSpec((2, _D, _D), lambda i: (0, 0, 0)),
            pl.BlockSpec((_D, _D), lambda i: (0, 0)),
            pl.BlockSpec((_D,), lambda i: (0,)),
            pl.BlockSpec((_D, _D), lambda i: (0, 0)),
            pl.BlockSpec((_D,), lambda i: (0,)),
        ],
        out_specs=pl.BlockSpec((_BE, _D), lambda i: (i, 0)),
        out_shape=jax.ShapeDtypeStruct((_N_EDGES, _D), jnp.float32),
    )(Z, w0, w1, Wv, W1, b1, W2, b2)
    return out


# speedup vs baseline: 1.0118x; 1.0118x over previous
import functools
import math

import jax
import jax.numpy as jnp
import numpy as np
from jax import lax
from jax.experimental import pallas as pl
from jax.experimental.pallas import tpu as pltpu
from jax.experimental.pallas import tpu_sc as plsc

_N_NODES = 10000
_N_EDGES = 320000
_D = 128

_BE = 12288
_GRID = -(-_N_EDGES // _BE)

_NSUB = 16
_CH = _N_EDGES // _NSUB
_NCHUNK = _CH // 16
_NPAD = 10240
_CPT = _NPAD // _NSUB


_F32 = jnp.float32
_DN_K1 = (((0,), (0,)), ((), ()))
_DN_RT = (((1,), (1,)), ((), ()))


def _att_body(z_ref, wq_ref, wk_ref, o0_ref, o1_ref, mx_ref):
    i = pl.program_id(0)
    z = z_ref[...]
    scale = 1.0 / math.sqrt(_D)
    a0w = lax.dot_general(wq_ref[0], wk_ref[0], _DN_K1,
                          preferred_element_type=_F32) * scale
    a1w = lax.dot_general(wq_ref[1], wk_ref[1], _DN_K1,
                          preferred_element_type=_F32) * scale
    p = jnp.concatenate([a0w, a1w], axis=1)
    t = jnp.dot(z, p, preferred_element_type=_F32)
    zz = jnp.concatenate([z, z], axis=1)
    hsel = lax.broadcasted_iota(jnp.int32, (2 * _D, 2), 0) // _D
    csel = lax.broadcasted_iota(jnp.int32, (2 * _D, 2), 1)
    sel = jnp.where(hsel == csel, 1.0, 0.0).astype(_F32)
    a01 = jnp.dot(t * zz, sel, preferred_element_type=_F32)
    a_t = a01.T
    o0_ref[...] = a_t[0]
    o1_ref[...] = a_t[1]
    rows = lax.broadcasted_iota(jnp.int32, (_BE, 2), 0)
    valid = _N_EDGES - i * _BE
    a01m = jnp.where(rows < valid, a01, -3.0e38)
    mfull = jnp.full((1, 1, _D), jnp.max(a01m), jnp.float32)

    @pl.when(i == 0)
    def _():
        mx_ref[...] = mfull

    @pl.when(i > 0)
    def _():
        mx_ref[...] = jnp.maximum(mx_ref[...], mfull)


def _out_body(z_ref, w0_ref, w1_ref, wv_ref, w1w_ref, b1_ref, w2w_ref,
              b2_ref, o_ref):
    z = z_ref[...]
    v0 = lax.dot_general(z, wv_ref[0], _DN_RT, preferred_element_type=_F32)
    v1 = lax.dot_general(z, wv_ref[1], _DN_RT, preferred_element_type=_F32)
    w_t = jnp.stack([w0_ref[...], w1_ref[...]], axis=0).T
    zu = w_t[:, 0:1] * v0 + w_t[:, 1:2] * v1
    h = lax.dot_general(zu, w1w_ref[...], _DN_RT, preferred_element_type=_F32)
    h = h + b1_ref[...][None, :]
    h = h * jax.nn.sigmoid(h)
    o = lax.dot_general(h, w2w_ref[...], _DN_RT, preferred_element_type=_F32)
    o_ref[...] = o + b2_ref[...][None, :]


def _sc_softmax_body(att0_hbm, att1_hbm, row_hbm, gmax_hbm,
                     w0_hbm, w1_hbm,
                     att_v, idx_v, denom_v, red_v, tot_v, gmax_v,
                     partials_sh, total_sh):
    c = lax.axis_index("c")
    s = lax.axis_index("s")
    base = s * _CH

    pltpu.sync_copy(gmax_hbm.at[0, 0, pl.ds(0, 16)], gmax_v)

    @pl.when(c == 0)
    def _():
        pltpu.sync_copy(att0_hbm.at[pl.ds(base, _CH)], att_v)

    @pl.when(c == 1)
    def _():
        pltpu.sync_copy(att1_hbm.at[pl.ds(base, _CH)], att_v)

    pltpu.sync_copy(row_hbm.at[pl.ds(base, _CH)], idx_v)

    @plsc.parallel_loop(0, _NPAD, step=16)
    def _(i):
        denom_v[pl.ds(pl.multiple_of(i, 16), 16)] = jnp.zeros(
            (16,), jnp.float32)

    gm = gmax_v[...]

    @plsc.parallel_loop(0, _CH, step=16)
    def _(i):
        off = pl.multiple_of(i, 16)
        idxv = idx_v[pl.ds(off, 16)]
        ev = jnp.exp(att_v[pl.ds(off, 16)] - gm)
        att_v[pl.ds(off, 16)] = ev
        plsc.addupdate_scatter(denom_v, [idxv], ev)

    pltpu.sync_copy(denom_v, partials_sh.at[s])
    plsc.subcore_barrier()
    colbase = s * _CPT
    pltpu.sync_copy(partials_sh.at[:, pl.ds(colbase, _CPT)], red_v)

    @plsc.parallel_loop(0, _CPT, step=16)
    def _(j):
        off = pl.multiple_of(j, 16)
        acc = red_v[0, pl.ds(off, 16)]
        for r in range(1, _NSUB):
            acc = acc + red_v[r, pl.ds(off, 16)]
        tot_v[pl.ds(off, 16)] = 1.0 / acc
    pltpu.sync_copy(tot_v, total_sh.at[pl.ds(colbase, _CPT)])
    plsc.subcore_barrier()
    pltpu.sync_copy(total_sh, denom_v)

    @plsc.parallel_loop(0, _CH, step=16)
    def _(i):
        off = pl.multiple_of(i, 16)
        idxv = idx_v[pl.ds(off, 16)]
        ev = att_v[pl.ds(off, 16)]
        dv = plsc.load_gather(denom_v, [idxv])
        att_v[pl.ds(off, 16)] = ev * dv

    @pl.when(c == 0)
    def _():
        pltpu.sync_copy(att_v, w0_hbm.at[pl.ds(base, _CH)])

    @pl.when(c == 1)
    def _():
        pltpu.sync_copy(att_v, w1_hbm.at[pl.ds(base, _CH)])


@functools.cache
def _sc_softmax():
  return functools.partial(
    pl.kernel,
    out_type=[jax.ShapeDtypeStruct((_N_EDGES,), jnp.float32),
              jax.ShapeDtypeStruct((_N_EDGES,), jnp.float32)],
    mesh=plsc.VectorSubcoreMesh(core_axis_name="c", subcore_axis_name="s",
                                num_cores=2, num_subcores=_NSUB),
    compiler_params=pltpu.CompilerParams(needs_layout_passes=False),
    scratch_types=[
        pltpu.VMEM((_CH,), jnp.float32),
        pltpu.VMEM((_CH,), jnp.int32),
        pltpu.VMEM((_NPAD,), jnp.float32),
        pltpu.VMEM((_NSUB, _CPT), jnp.float32),
        pltpu.VMEM((_CPT,), jnp.float32),
        pltpu.VMEM((16,), jnp.float32),
        pltpu.VMEM_SHARED((_NSUB, _NPAD), jnp.float32),
        pltpu.VMEM_SHARED((_NPAD,), jnp.float32),
    ],
  )(_sc_softmax_body)


def kernel(Z, edges, Wq, Wk, Wv, W1, b1, W2, b2):
    att0, att1, mx = pl.pallas_call(
        _att_body,
        grid=(_GRID,),
        in_specs=[
            pl.BlockSpec((_BE, _D), lambda i: (i, 0)),
            pl.BlockSpec((2, _D, _D), lambda i: (0, 0, 0)),
            pl.BlockSpec((2, _D, _D), lambda i: (0, 0, 0)),
        ],
        out_specs=[
            pl.BlockSpec((_BE,), lambda i: (i,)),
            pl.BlockSpec((_BE,), lambda i: (i,)),
            pl.BlockSpec((1, 1, _D), lambda i: (0, 0, 0)),
        ],
        out_shape=[
            jax.ShapeDtypeStruct((_N_EDGES,), jnp.float32),
            jax.ShapeDtypeStruct((_N_EDGES,), jnp.float32),
            jax.ShapeDtypeStruct((1, 1, _D), jnp.float32),
        ],
    )(Z, Wq, Wk)

    row2e = edges.astype(jnp.int32).reshape(2 * _N_EDGES)
    w0, w1 = _sc_softmax()(att0, att1, row2e, mx)

    out = pl.pallas_call(
        _out_body,
        grid=(_GRID,),
        in_specs=[
            pl.BlockSpec((_BE, _D), lambda i: (i, 0)),
            pl.BlockSpec((_BE,), lambda i: (i,)),
            pl.BlockSpec((_BE,), lambda i: (i,)),
            pl.BlockSpec((2, _D, _D), lambda i: (0, 0, 0)),
            pl.BlockSpec((_D, _D), lambda i: (0, 0)),
            pl.BlockSpec((_D,), lambda i: (0,)),
            pl.BlockSpec((_D, _D), lambda i: (0, 0)),
            pl.BlockSpec((_D,), lambda i: (0,)),
        ],
        out_specs=pl.BlockSpec((_BE, _D), lambda i: (i, 0)),
        out_shape=jax.ShapeDtypeStruct((_N_EDGES, _D), jnp.float32),
    )(Z, w0, w1, Wv, W1, b1, W2, b2)
    return out
